# Initial kernel scaffold; baseline (speedup 1.0000x reference)
import jax
import jax.numpy as jnp
from jax.experimental import pallas as pl


def kernel(latent_Z, alpha, sampling_weights, edge_index, sample_size):
    raise NotImplementedError("write your pallas kernel here")


# speedup vs baseline: 78.8660x; 78.8660x over previous
import functools

import jax
import jax.numpy as jnp
from jax import lax
from jax.experimental import pallas as pl
from jax.experimental.pallas import tpu as pltpu
from jax.experimental.pallas import tpu_sc as plsc

N_NODES = 100000
N_EDGES = 1600000
LATENT_DIM = 32
S = 1024
EPS = 1e-6

try:
    _info = plsc.get_sparse_core_info()
    NC, NS = _info.num_cores, _info.num_subcores
except Exception:
    NC, NS = 2, 16
NW = NC * NS
E_PER = N_EDGES // NW
E_BLK = 10000
N_BLK = E_PER // E_BLK
N_CHUNK = E_BLK // 16


def _nsqrt(x):
    i = lax.bitcast_convert_type(x, jnp.int32)
    y = lax.bitcast_convert_type((i >> 1) + 0x1FBD1DF5, jnp.float32)
    for _ in range(4):
        y = 0.5 * (y + x / y)
    return y


def _edge_body(src_hbm, dst_hbm, tbl_hbm, zt_hbm, dsum_out, csum_out,
               tbl_v, zt_v, src_v, dst_v, out_v):
    cid = lax.axis_index("c")
    sid = lax.axis_index("s")
    wid = sid * NC + cid
    pltpu.sync_copy(tbl_hbm, tbl_v)
    pltpu.sync_copy(zt_hbm, zt_v)
    base = wid * E_PER

    zero16 = jnp.zeros((16,), jnp.float32)

    def chunk(c, carry):
        dacc, cacc = carry
        s16 = src_v[pl.ds(c * 16, 16)]
        d16 = dst_v[pl.ds(c * 16, 16)]
        pw = plsc.load_gather(tbl_v, [lax.shift_right_logical(s16, 1)])
        p = lax.shift_right_logical(pw, (s16 & 1) << 4) & 0xFFFF
        qw = plsc.load_gather(tbl_v, [lax.shift_right_logical(d16, 1)])
        q = lax.shift_right_logical(qw, (d16 & 1) << 4) & 0xFFFF
        valid = (p < S) & (q < S)
        nv = plsc.all_reduce_population_count(valid)
        nvs = lax.reduce_max(nv, axes=(0,))

        def heavy(_):
            pc = jnp.where(valid, p, 0)
            qc = jnp.where(valid, q, 0)
            a2 = zero16
            for d in range(LATENT_DIM):
                zp = plsc.load_gather(zt_v, [pc + d * S])
                zq = plsc.load_gather(zt_v, [qc + d * S])
                df = zp - zq + EPS
                a2 = a2 + df * df
            dist = _nsqrt(a2)
            return (jnp.where(valid, dist, 0.0),
                    jnp.where(valid, 1.0, 0.0))

        def light(_):
            return (zero16, zero16)

        dd, cc = lax.cond(nvs > 0, heavy, light, 0)
        return (dacc + dd, cacc + cc)

    def blk(b, carry):
        off = base + b * E_BLK
        pltpu.sync_copy(src_hbm.at[pl.ds(off, E_BLK)], src_v)
        pltpu.sync_copy(dst_hbm.at[pl.ds(off, E_BLK)], dst_v)
        return lax.fori_loop(0, N_CHUNK, chunk, carry)

    dacc, cacc = lax.fori_loop(0, N_BLK, blk, (zero16, zero16))
    out_v[...] = dacc
    pltpu.sync_copy(out_v, dsum_out.at[wid])
    out_v[...] = cacc
    pltpu.sync_copy(out_v, csum_out.at[wid])


def _make_edge_kernel():
    return functools.partial(
        pl.kernel,
        mesh=plsc.VectorSubcoreMesh(core_axis_name="c", subcore_axis_name="s"),
        compiler_params=pltpu.CompilerParams(needs_layout_passes=False),
        out_type=[
            jax.ShapeDtypeStruct((NW, 16), jnp.float32),
            jax.ShapeDtypeStruct((NW, 16), jnp.float32),
        ],
        scratch_types=[
            pltpu.VMEM((N_NODES // 2,), jnp.int32),
            pltpu.VMEM((S * LATENT_DIM,), jnp.float32),
            pltpu.VMEM((E_BLK,), jnp.int32),
            pltpu.VMEM((E_BLK,), jnp.int32),
            pltpu.VMEM((16,), jnp.float32),
        ],
    )(_edge_body)


def _pdist_body(zi_ref, za_ref, out_ref):
    zi = zi_ref[...]
    za = za_ref[...]
    acc = jnp.zeros((128, S), jnp.float32)
    for d in range(LATENT_DIM):
        df = zi[:, d][:, None] - za[:, d][None, :] + EPS
        acc = acc + df * df
    mat = jnp.exp(-jnp.sqrt(acc))
    out_ref[...] = jnp.full((1, 1, 128), jnp.sum(mat), jnp.float32)


_pdist_call = pl.pallas_call(
    _pdist_body,
    grid=(8,),
    in_specs=[
        pl.BlockSpec((128, LATENT_DIM), lambda i: (i, 0)),
        pl.BlockSpec((S, LATENT_DIM), lambda i: (0, 0)),
    ],
    out_specs=pl.BlockSpec((1, 1, 128), lambda i: (i, 0, 0)),
    out_shape=jax.ShapeDtypeStruct((8, 1, 128), jnp.float32),
)


def kernel(latent_Z, alpha, sampling_weights, edge_index, sample_size):
    skey = jax.random.key(42)
    u = jax.random.uniform(skey, sampling_weights.shape, minval=1e-9, maxval=1.0)
    gumbel = -jnp.log(-jnp.log(u))
    scores = jnp.log(sampling_weights) + gumbel
    _, sample_idx = lax.top_k(scores, S)

    tbl = jnp.full((N_NODES,), 0xFFFF, jnp.int32)
    tbl = tbl.at[sample_idx].set(jnp.arange(S, dtype=jnp.int32))
    tbl_packed = tbl[0::2] | (tbl[1::2] << 16)

    Zs = latent_Z[sample_idx]
    zt_flat = Zs.T.reshape(-1)

    blocksums = _pdist_call(Zs, Zs)
    mat_total = jnp.sum(blocksums[:, 0, 0])
    diag_const = jnp.exp(-jnp.sqrt(jnp.sum(jnp.full((LATENT_DIM,), EPS,
                                                    jnp.float32) ** 2)))
    offdiag = mat_total - S * diag_const
    e1 = jnp.exp(jnp.float32(1.0))
    z_pdist1 = jnp.exp(alpha[0]) * (0.5 * (e1 * e1 * offdiag))

    dsum, csum = _make_edge_kernel()(
        edge_index[0], edge_index[1], tbl_packed, zt_flat)
    sum_d = jnp.sum(dsum)
    n_valid = jnp.sum(csum)
    z_pdist2 = alpha[0] * n_valid - sum_d

    out = z_pdist2 - z_pdist1
    return jnp.full((1, 1), out, jnp.float32)


# speedup vs baseline: 116.5468x; 1.4778x over previous
import functools

import jax
import jax.numpy as jnp
from jax import lax
from jax.experimental import pallas as pl
from jax.experimental.pallas import tpu as pltpu
from jax.experimental.pallas import tpu_sc as plsc

N_NODES = 100000
N_EDGES = 1600000
LATENT_DIM = 32
S = 1024
EPS = 1e-6
SENT = 0xFFFF

ROWS = 782
NP = ROWS * 128

try:
    _info = plsc.get_sparse_core_info()
    NC, NS = _info.num_cores, _info.num_subcores
except Exception:
    NC, NS = 2, 16
NW = NC * NS
E_PER = N_EDGES // NW
E_BLK = 10000
N_BLK = E_PER // E_BLK
N_CHUNK = E_BLK // 16

NODES_PER = 3136
NODES_PAD = NODES_PER * NW
SIDX_BUF = 1040


def _nsqrt(x):
    i = lax.bitcast_convert_type(x, jnp.int32)
    y = lax.bitcast_convert_type((i >> 1) + 0x1FBD1DF5, jnp.float32)
    for _ in range(4):
        y = 0.5 * (y + x / y)
    return y


def _cumsum_lanes(x):
    for k in (1, 2, 4, 8, 16, 32, 64):
        x = x + jnp.pad(x, ((0, 0), (k, 0)))[:, :-k]
    return x


def _cumsum_rows(x):
    for k in (1, 2, 4, 8, 16, 32, 64, 128, 256, 512):
        x = x + jnp.pad(x, ((k, 0), (0, 0)))[:-k, :]
    return x


def _thresh_body(w_ref, g_ref, tbl_ref):
    s = jnp.log(w_ref[...]) + g_ref[...]
    i = lax.bitcast_convert_type(s, jnp.int32)
    key = jnp.where(i >= 0, i, i ^ 0x7FFFFFFF)
    cnt_pos = jnp.sum((key >= 0).astype(jnp.int32))
    prefix0 = jnp.where(cnt_pos >= S, 0, jnp.int32(-2147483648))

    def bit_body(t, prefix):
        cand = prefix | lax.shift_left(jnp.int32(1), 30 - t)
        c = jnp.sum((key >= cand).astype(jnp.int32))
        return jnp.where(c >= S, cand, prefix)

    v = lax.fori_loop(0, 31, bit_body, prefix0)
    c_gt = jnp.sum((key > v).astype(jnp.int32))
    need = S - c_gt
    eq32 = (key == v).astype(jnp.int32)
    rowc = _cumsum_lanes(eq32)
    rowtot = rowc[:, 127:128]
    rowoff = _cumsum_rows(rowtot) - rowtot
    excl_eq = rowc - eq32 + rowoff
    mask = (key > v) | ((eq32 > 0) & (excl_eq < need))
    m32 = mask.astype(jnp.int32)
    rc2 = _cumsum_lanes(m32)
    rt2 = rc2[:, 127:128]
    ro2 = _cumsum_rows(rt2) - rt2
    pos = rc2 - m32 + ro2
    tbl_ref[...] = jnp.where(mask, pos, SENT)


_thresh_call = pl.pallas_call(
    _thresh_body,
    in_specs=[
        pl.BlockSpec((ROWS, 128), lambda: (0, 0)),
        pl.BlockSpec((ROWS, 128), lambda: (0, 0)),
    ],
    out_specs=pl.BlockSpec((ROWS, 128), lambda: (0, 0)),
    out_shape=jax.ShapeDtypeStruct((ROWS, 128), jnp.int32),
)


def _sidx_body(tbl_hbm, out_hbm, rng_v, sbuf_v):
    cid = lax.axis_index("c")
    sid = lax.axis_index("s")
    wid = sid * NC + cid
    base = wid * NODES_PER
    pltpu.sync_copy(tbl_hbm.at[pl.ds(base, NODES_PER)], rng_v)
    zeros16 = jnp.zeros((16,), jnp.int32)

    def zinit(c, _):
        sbuf_v[pl.ds(c * 16, 16)] = zeros16
        return 0

    lax.fori_loop(0, SIDX_BUF // 16, zinit, 0)

    def chunk(c, _):
        tv = rng_v[pl.ds(c * 16, 16)]
        nodes = base + c * 16 + lax.iota(jnp.int32, 16)
        valid = tv < S
        slot = jnp.where(valid, tv, S)
        plsc.store_scatter(sbuf_v, [slot], nodes, mask=valid)
        return 0

    lax.fori_loop(0, NODES_PER // 16, chunk, 0)
    pltpu.sync_copy(sbuf_v, out_hbm.at[wid])


def _make_sidx_kernel():
    return functools.partial(
        pl.kernel,
        mesh=plsc.VectorSubcoreMesh(core_axis_name="c", subcore_axis_name="s"),
        compiler_params=pltpu.CompilerParams(needs_layout_passes=False),
        out_type=jax.ShapeDtypeStruct((NW, SIDX_BUF), jnp.int32),
        scratch_types=[
            pltpu.VMEM((NODES_PER,), jnp.int32),
            pltpu.VMEM((SIDX_BUF,), jnp.int32),
        ],
    )(_sidx_body)


def _edge_body(src_hbm, dst_hbm, tbl_hbm, zt_hbm, dsum_out, csum_out,
               tbl_v, zt_v, src_v, dst_v, out_v):
    cid = lax.axis_index("c")
    sid = lax.axis_index("s")
    wid = sid * NC + cid
    pltpu.sync_copy(tbl_hbm, tbl_v)
    pltpu.sync_copy(zt_hbm, zt_v)
    base = wid * E_PER

    zero16 = jnp.zeros((16,), jnp.float32)

    def chunk(c, carry):
        dacc, cacc = carry
        s16 = src_v[pl.ds(c * 16, 16)]
        d16 = dst_v[pl.ds(c * 16, 16)]
        pw = plsc.load_gather(tbl_v, [lax.shift_right_logical(s16, 1)])
        p = lax.shift_right_logical(pw, (s16 & 1) << 4) & 0xFFFF
        qw = plsc.load_gather(tbl_v, [lax.shift_right_logical(d16, 1)])
        q = lax.shift_right_logical(qw, (d16 & 1) << 4) & 0xFFFF
        valid = (p < S) & (q < S)
        nv = plsc.all_reduce_population_count(valid)
        nvs = lax.reduce_max(nv, axes=(0,))

        def heavy(_):
            pc = jnp.where(valid, p, 0)
            qc = jnp.where(valid, q, 0)
            a2 = zero16
            for d in range(LATENT_DIM):
                zp = plsc.load_gather(zt_v, [pc + d * S])
                zq = plsc.load_gather(zt_v, [qc + d * S])
                df = zp - zq + EPS
                a2 = a2 + df * df
            dist = _nsqrt(a2)
            return (jnp.where(valid, dist, 0.0),
                    jnp.where(valid, 1.0, 0.0))

        def light(_):
            return (zero16, zero16)

        dd, cc = lax.cond(nvs > 0, heavy, light, 0)
        return (dacc + dd, cacc + cc)

    def blk(b, carry):
        off = base + b * E_BLK
        pltpu.sync_copy(src_hbm.at[pl.ds(off, E_BLK)], src_v)
        pltpu.sync_copy(dst_hbm.at[pl.ds(off, E_BLK)], dst_v)
        return lax.fori_loop(0, N_CHUNK, chunk, carry)

    dacc, cacc = lax.fori_loop(0, N_BLK, blk, (zero16, zero16))
    out_v[...] = dacc
    pltpu.sync_copy(out_v, dsum_out.at[wid])
    out_v[...] = cacc
    pltpu.sync_copy(out_v, csum_out.at[wid])


def _make_edge_kernel():
    return functools.partial(
        pl.kernel,
        mesh=plsc.VectorSubcoreMesh(core_axis_name="c", subcore_axis_name="s"),
        compiler_params=pltpu.CompilerParams(needs_layout_passes=False),
        out_type=[
            jax.ShapeDtypeStruct((NW, 16), jnp.float32),
            jax.ShapeDtypeStruct((NW, 16), jnp.float32),
        ],
        scratch_types=[
            pltpu.VMEM((N_NODES // 2,), jnp.int32),
            pltpu.VMEM((S * LATENT_DIM,), jnp.float32),
            pltpu.VMEM((E_BLK,), jnp.int32),
            pltpu.VMEM((E_BLK,), jnp.int32),
            pltpu.VMEM((16,), jnp.float32),
        ],
    )(_edge_body)


def _pdist_body(zi_ref, za_ref, out_ref):
    zi = zi_ref[...]
    za = za_ref[...]
    acc = jnp.zeros((128, S), jnp.float32)
    for d in range(LATENT_DIM):
        df = zi[:, d][:, None] - za[:, d][None, :] + EPS
        acc = acc + df * df
    mat = jnp.exp(-jnp.sqrt(acc))
    out_ref[...] = jnp.full((1, 1, 128), jnp.sum(mat), jnp.float32)


_pdist_call = pl.pallas_call(
    _pdist_body,
    grid=(8,),
    in_specs=[
        pl.BlockSpec((128, LATENT_DIM), lambda i: (i, 0)),
        pl.BlockSpec((S, LATENT_DIM), lambda i: (0, 0)),
    ],
    out_specs=pl.BlockSpec((1, 1, 128), lambda i: (i, 0, 0)),
    out_shape=jax.ShapeDtypeStruct((8, 1, 128), jnp.float32),
)


def kernel(latent_Z, alpha, sampling_weights, edge_index, sample_size):
    skey = jax.random.key(42)
    u = jax.random.uniform(skey, sampling_weights.shape, minval=1e-9, maxval=1.0)
    gumbel = -jnp.log(-jnp.log(u))

    w_pad = jnp.pad(sampling_weights, (0, NP - N_NODES),
                    constant_values=1e-30).reshape(ROWS, 128)
    g_pad = jnp.pad(gumbel, (0, NP - N_NODES)).reshape(ROWS, 128)

    tbl2d = _thresh_call(w_pad, g_pad)
    tbl = tbl2d.reshape(-1)[:N_NODES]
    tbl_packed = tbl[0::2] | (tbl[1::2] << 16)

    tbl_sc = jnp.pad(tbl, (0, NODES_PAD - N_NODES), constant_values=SENT)
    sidx_parts = _make_sidx_kernel()(tbl_sc)
    sample_idx = jnp.sum(sidx_parts, axis=0)[:S]
    Zs = latent_Z[sample_idx]
    zt_flat = Zs.T.reshape(-1)

    blocksums = _pdist_call(Zs, Zs)
    mat_total = jnp.sum(blocksums[:, 0, 0])
    diag_const = jnp.exp(-jnp.sqrt(jnp.sum(jnp.full((LATENT_DIM,), EPS,
                                                    jnp.float32) ** 2)))
    offdiag = mat_total - S * diag_const
    e1 = jnp.exp(jnp.float32(1.0))
    z_pdist1 = jnp.exp(alpha[0]) * (0.5 * (e1 * e1 * offdiag))

    dsum, csum = _make_edge_kernel()(
        edge_index[0], edge_index[1], tbl_packed, zt_flat)
    sum_d = jnp.sum(dsum)
    n_valid = jnp.sum(csum)
    z_pdist2 = alpha[0] * n_valid - sum_d

    out = z_pdist2 - z_pdist1
    return jnp.full((1, 1), out, jnp.float32)


# speedup vs baseline: 145.5859x; 1.2492x over previous
import functools

import jax
import jax.numpy as jnp
from jax import lax
from jax.experimental import pallas as pl
from jax.experimental.pallas import tpu as pltpu
from jax.experimental.pallas import tpu_sc as plsc

N_NODES = 100000
N_EDGES = 1600000
LATENT_DIM = 32
S = 1024
EPS = 1e-6
SENT = 0xFFFF

ROWS = 782
NP = ROWS * 128

try:
    _info = plsc.get_sparse_core_info()
    NC, NS = _info.num_cores, _info.num_subcores
except Exception:
    NC, NS = 2, 16
NW = NC * NS
E_PER = N_EDGES // NW
E_BLK = 10000
N_BLK = E_PER // E_BLK
N_CHUNK = E_BLK // 16

NODES_PER = 3136
NODES_PAD = NODES_PER * NW
SIDX_BUF = 1040


def _nsqrt(x):
    i = lax.bitcast_convert_type(x, jnp.int32)
    y = lax.bitcast_convert_type((i >> 1) + 0x1FBD1DF5, jnp.float32)
    for _ in range(4):
        y = 0.5 * (y + x / y)
    return y


def _cumsum_lanes(x):
    for k in (1, 2, 4, 8, 16, 32, 64):
        x = x + jnp.pad(x, ((0, 0), (k, 0)))[:, :-k]
    return x


def _cumsum_rows(x):
    for k in (1, 2, 4, 8, 16, 32, 64, 128, 256, 512):
        x = x + jnp.pad(x, ((k, 0), (0, 0)))[:-k, :]
    return x


def _thresh_body(w_ref, g_ref, tbl_ref):
    s = jnp.log(w_ref[...]) + g_ref[...]
    i = lax.bitcast_convert_type(s, jnp.int32)
    key = jnp.where(i >= 0, i, i ^ 0x7FFFFFFF)
    cnt_pos = jnp.sum((key >= 0).astype(jnp.int32))
    prefix0 = jnp.where(cnt_pos >= S, 0, jnp.int32(-2147483648))

    def bit_body(t, prefix):
        cand = prefix | lax.shift_left(jnp.int32(1), 30 - t)
        c = jnp.sum((key >= cand).astype(jnp.int32))
        return jnp.where(c >= S, cand, prefix)

    v = lax.fori_loop(0, 31, bit_body, prefix0)
    c_gt = jnp.sum((key > v).astype(jnp.int32))
    need = S - c_gt
    eq32 = (key == v).astype(jnp.int32)
    rowc = _cumsum_lanes(eq32)
    rowtot = rowc[:, 127:128]
    rowoff = _cumsum_rows(rowtot) - rowtot
    excl_eq = rowc - eq32 + rowoff
    mask = (key > v) | ((eq32 > 0) & (excl_eq < need))
    m32 = mask.astype(jnp.int32)
    rc2 = _cumsum_lanes(m32)
    rt2 = rc2[:, 127:128]
    ro2 = _cumsum_rows(rt2) - rt2
    pos = rc2 - m32 + ro2
    tbl_ref[...] = jnp.where(mask, pos, SENT)


_thresh_call = pl.pallas_call(
    _thresh_body,
    in_specs=[
        pl.BlockSpec((ROWS, 128), lambda: (0, 0)),
        pl.BlockSpec((ROWS, 128), lambda: (0, 0)),
    ],
    out_specs=pl.BlockSpec((ROWS, 128), lambda: (0, 0)),
    out_shape=jax.ShapeDtypeStruct((ROWS, 128), jnp.int32),
)


def _sidx_body(tbl_hbm, out_hbm, rng_v, sbuf_v):
    cid = lax.axis_index("c")
    sid = lax.axis_index("s")
    wid = sid * NC + cid
    base = wid * NODES_PER
    pltpu.sync_copy(tbl_hbm.at[pl.ds(base, NODES_PER)], rng_v)
    zeros16 = jnp.zeros((16,), jnp.int32)

    def zinit(c, _):
        sbuf_v[pl.ds(c * 16, 16)] = zeros16
        return 0

    lax.fori_loop(0, SIDX_BUF // 16, zinit, 0)

    def chunk(c, _):
        tv = rng_v[pl.ds(c * 16, 16)]
        nodes = base + c * 16 + lax.iota(jnp.int32, 16)
        valid = tv < S
        slot = jnp.where(valid, tv, S)
        plsc.store_scatter(sbuf_v, [slot], nodes, mask=valid)
        return 0

    lax.fori_loop(0, NODES_PER // 16, chunk, 0)
    pltpu.sync_copy(sbuf_v, out_hbm.at[wid])


def _make_sidx_kernel():
    return functools.partial(
        pl.kernel,
        mesh=plsc.VectorSubcoreMesh(core_axis_name="c", subcore_axis_name="s"),
        compiler_params=pltpu.CompilerParams(needs_layout_passes=False),
        out_type=jax.ShapeDtypeStruct((NW, SIDX_BUF), jnp.int32),
        scratch_types=[
            pltpu.VMEM((NODES_PER,), jnp.int32),
            pltpu.VMEM((SIDX_BUF,), jnp.int32),
        ],
    )(_sidx_body)


def _edge_body(src_hbm, dst_hbm, tbl_hbm, zt_hbm, dsum_out, csum_out,
               tbl_v, zt_v, s0, d0, s1, d1, out_v, sem0, sem1):
    cid = lax.axis_index("c")
    sid = lax.axis_index("s")
    wid = sid * NC + cid
    pltpu.sync_copy(tbl_hbm, tbl_v)
    pltpu.sync_copy(zt_hbm, zt_v)
    base = wid * E_PER

    zero16 = jnp.zeros((16,), jnp.float32)

    def lookup(sv, dv, off):
        s16 = sv[pl.ds(off, 16)]
        d16 = dv[pl.ds(off, 16)]
        pw = plsc.load_gather(tbl_v, [lax.shift_right_logical(s16, 1)])
        p = lax.shift_right_logical(pw, (s16 & 1) << 4) & 0xFFFF
        qw = plsc.load_gather(tbl_v, [lax.shift_right_logical(d16, 1)])
        q = lax.shift_right_logical(qw, (d16 & 1) << 4) & 0xFFFF
        return p, q, (p < S) & (q < S)

    def pair_dist(p, q, valid):
        pc = jnp.where(valid, p, 0)
        qc = jnp.where(valid, q, 0)
        a2 = zero16
        for d in range(LATENT_DIM):
            zp = plsc.load_gather(zt_v, [pc + d * S])
            zq = plsc.load_gather(zt_v, [qc + d * S])
            df = zp - zq + EPS
            a2 = a2 + df * df
        dist = _nsqrt(a2)
        return (jnp.where(valid, dist, 0.0), jnp.where(valid, 1.0, 0.0))

    def make_chunk2(sv, dv):
        def chunk2(c, carry):
            dacc, cacc = carry
            pa, qa, va = lookup(sv, dv, c * 32)
            pb, qb, vb = lookup(sv, dv, c * 32 + 16)
            nv = (plsc.all_reduce_population_count(va)
                  + plsc.all_reduce_population_count(vb))
            nvs = lax.reduce_max(nv, axes=(0,))

            def heavy(_):
                da, ca = pair_dist(pa, qa, va)
                db, cb = pair_dist(pb, qb, vb)
                return (da + db, ca + cb)

            def light(_):
                return (zero16, zero16)

            dd, cc = lax.cond(nvs > 0, heavy, light, 0)
            return (dacc + dd, cacc + cc)
        return chunk2

    def tail16(sv, dv, carry):
        dacc, cacc = carry
        p, q, valid = lookup(sv, dv, E_BLK - 16)
        nv = plsc.all_reduce_population_count(valid)
        nvs = lax.reduce_max(nv, axes=(0,))

        def heavy(_):
            return pair_dist(p, q, valid)

        def light(_):
            return (zero16, zero16)

        dd, cc = lax.cond(nvs > 0, heavy, light, 0)
        return (dacc + dd, cacc + cc)

    bufs = ((s0, d0, sem0), (s1, d1, sem1))
    pltpu.async_copy(src_hbm.at[pl.ds(base, E_BLK)], s0, sem0)
    pltpu.async_copy(dst_hbm.at[pl.ds(base, E_BLK)], d0, sem0)
    carry = (zero16, zero16)
    for b in range(N_BLK):
        sv, dv, sem = bufs[b % 2]
        if b + 1 < N_BLK:
            nsv, ndv, nsem = bufs[(b + 1) % 2]
            noff = base + (b + 1) * E_BLK
            pltpu.async_copy(src_hbm.at[pl.ds(noff, E_BLK)], nsv, nsem)
            pltpu.async_copy(dst_hbm.at[pl.ds(noff, E_BLK)], ndv, nsem)
        off = base + b * E_BLK
        pltpu.make_async_copy(src_hbm.at[pl.ds(off, E_BLK)], sv, sem).wait()
        pltpu.make_async_copy(dst_hbm.at[pl.ds(off, E_BLK)], dv, sem).wait()
        carry = lax.fori_loop(0, (E_BLK - 16) // 32, make_chunk2(sv, dv), carry)
        carry = tail16(sv, dv, carry)
    dacc, cacc = carry
    out_v[...] = dacc
    pltpu.sync_copy(out_v, dsum_out.at[wid])
    out_v[...] = cacc
    pltpu.sync_copy(out_v, csum_out.at[wid])


def _make_edge_kernel():
    return functools.partial(
        pl.kernel,
        mesh=plsc.VectorSubcoreMesh(core_axis_name="c", subcore_axis_name="s"),
        compiler_params=pltpu.CompilerParams(needs_layout_passes=False),
        out_type=[
            jax.ShapeDtypeStruct((NW, 16), jnp.float32),
            jax.ShapeDtypeStruct((NW, 16), jnp.float32),
        ],
        scratch_types=[
            pltpu.VMEM((N_NODES // 2,), jnp.int32),
            pltpu.VMEM((S * LATENT_DIM,), jnp.float32),
            pltpu.VMEM((E_BLK,), jnp.int32),
            pltpu.VMEM((E_BLK,), jnp.int32),
            pltpu.VMEM((E_BLK,), jnp.int32),
            pltpu.VMEM((E_BLK,), jnp.int32),
            pltpu.VMEM((16,), jnp.float32),
            pltpu.SemaphoreType.DMA,
            pltpu.SemaphoreType.DMA,
        ],
    )(_edge_body)


def _pdist_body(zi_ref, za_ref, out_ref):
    zi = zi_ref[...]
    za = za_ref[...]
    acc = jnp.zeros((128, S), jnp.float32)
    for d in range(LATENT_DIM):
        df = zi[:, d][:, None] - za[:, d][None, :] + EPS
        acc = acc + df * df
    mat = jnp.exp(-jnp.sqrt(acc))
    out_ref[...] = jnp.full((1, 1, 128), jnp.sum(mat), jnp.float32)


_pdist_call = pl.pallas_call(
    _pdist_body,
    grid=(8,),
    in_specs=[
        pl.BlockSpec((128, LATENT_DIM), lambda i: (i, 0)),
        pl.BlockSpec((S, LATENT_DIM), lambda i: (0, 0)),
    ],
    out_specs=pl.BlockSpec((1, 1, 128), lambda i: (i, 0, 0)),
    out_shape=jax.ShapeDtypeStruct((8, 1, 128), jnp.float32),
)


def kernel(latent_Z, alpha, sampling_weights, edge_index, sample_size):
    skey = jax.random.key(42)
    u = jax.random.uniform(skey, sampling_weights.shape, minval=1e-9, maxval=1.0)
    gumbel = -jnp.log(-jnp.log(u))

    w_pad = jnp.pad(sampling_weights, (0, NP - N_NODES),
                    constant_values=1e-30).reshape(ROWS, 128)
    g_pad = jnp.pad(gumbel, (0, NP - N_NODES)).reshape(ROWS, 128)

    tbl2d = _thresh_call(w_pad, g_pad)
    tbl = tbl2d.reshape(-1)[:N_NODES]
    tbl_packed = tbl[0::2] | (tbl[1::2] << 16)

    tbl_sc = jnp.pad(tbl, (0, NODES_PAD - N_NODES), constant_values=SENT)
    sidx_parts = _make_sidx_kernel()(tbl_sc)
    sample_idx = jnp.sum(sidx_parts, axis=0)[:S]
    Zs = latent_Z[sample_idx]
    zt_flat = Zs.T.reshape(-1)

    blocksums = _pdist_call(Zs, Zs)
    mat_total = jnp.sum(blocksums[:, 0, 0])
    diag_const = jnp.exp(-jnp.sqrt(jnp.sum(jnp.full((LATENT_DIM,), EPS,
                                                    jnp.float32) ** 2)))
    offdiag = mat_total - S * diag_const
    e1 = jnp.exp(jnp.float32(1.0))
    z_pdist1 = jnp.exp(alpha[0]) * (0.5 * (e1 * e1 * offdiag))

    dsum, csum = _make_edge_kernel()(
        edge_index[0], edge_index[1], tbl_packed, zt_flat)
    sum_d = jnp.sum(dsum)
    n_valid = jnp.sum(csum)
    z_pdist2 = alpha[0] * n_valid - sum_d

    out = z_pdist2 - z_pdist1
    return jnp.full((1, 1), out, jnp.float32)


# speedup vs baseline: 160.5683x; 1.1029x over previous
import functools

import numpy as np
import jax
import jax.numpy as jnp
from jax import lax
from jax.experimental import pallas as pl
from jax.experimental.pallas import tpu as pltpu
from jax.experimental.pallas import tpu_sc as plsc

N_NODES = 100000
N_EDGES = 1600000
LATENT_DIM = 32
S = 1024
EPS = 1e-6
SENT = 0xFFFF

ROWS = 782
NP = ROWS * 128

try:
    _info = plsc.get_sparse_core_info()
    NC, NS = _info.num_cores, _info.num_subcores
except Exception:
    NC, NS = 2, 16
NW = NC * NS
E_PER = N_EDGES // NW
E_BLK = 10000
N_BLK = E_PER // E_BLK
N_CHUNK = E_BLK // 16

NODES_PER = 3136
NODES_PAD = NODES_PER * NW
SIDX_BUF = 1040


def _nsqrt(x):
    i = lax.bitcast_convert_type(x, jnp.int32)
    y = lax.bitcast_convert_type((i >> 1) + 0x1FBD1DF5, jnp.float32)
    for _ in range(4):
        y = 0.5 * (y + x / y)
    return y


def _cumsum_lanes(x):
    for k in (1, 2, 4, 8, 16, 32, 64):
        x = x + jnp.pad(x, ((0, 0), (k, 0)))[:, :-k]
    return x


def _cumsum_rows(x):
    for k in (1, 2, 4, 8, 16, 32, 64, 128, 256, 512):
        x = x + jnp.pad(x, ((k, 0), (0, 0)))[:-k, :]
    return x


def _thresh_body(w_ref, g_ref, tbl_ref):
    s = jnp.log(w_ref[...]) + g_ref[...]
    i = lax.bitcast_convert_type(s, jnp.int32)
    key = jnp.where(i >= 0, i, i ^ 0x7FFFFFFF)
    cnt_pos = jnp.sum((key >= 0).astype(jnp.int32))
    prefix0 = jnp.where(cnt_pos >= S, 0, jnp.int32(-2147483648))

    def bit_body(t, prefix):
        cand = prefix | lax.shift_left(jnp.int32(1), 30 - t)
        c = jnp.sum((key >= cand).astype(jnp.int32))
        return jnp.where(c >= S, cand, prefix)

    v = lax.fori_loop(0, 31, bit_body, prefix0)
    c_gt = jnp.sum((key > v).astype(jnp.int32))
    need = S - c_gt
    eq32 = (key == v).astype(jnp.int32)
    rowc = _cumsum_lanes(eq32)
    rowtot = rowc[:, 127:128]
    rowoff = _cumsum_rows(rowtot) - rowtot
    excl_eq = rowc - eq32 + rowoff
    mask = (key > v) | ((eq32 > 0) & (excl_eq < need))
    m32 = mask.astype(jnp.int32)
    rc2 = _cumsum_lanes(m32)
    rt2 = rc2[:, 127:128]
    ro2 = _cumsum_rows(rt2) - rt2
    pos = rc2 - m32 + ro2
    tbl_ref[...] = jnp.where(mask, pos, SENT)


_thresh_call = pl.pallas_call(
    _thresh_body,
    in_specs=[
        pl.BlockSpec((ROWS, 128), lambda: (0, 0)),
        pl.BlockSpec((ROWS, 128), lambda: (0, 0)),
    ],
    out_specs=pl.BlockSpec((ROWS, 128), lambda: (0, 0)),
    out_shape=jax.ShapeDtypeStruct((ROWS, 128), jnp.int32),
)


def _sidx_body(tbl_hbm, out_hbm, rng_v, sbuf_v):
    cid = lax.axis_index("c")
    sid = lax.axis_index("s")
    wid = sid * NC + cid
    base = wid * NODES_PER
    pltpu.sync_copy(tbl_hbm.at[pl.ds(base, NODES_PER)], rng_v)
    zeros16 = jnp.zeros((16,), jnp.int32)

    def zinit(c, _):
        sbuf_v[pl.ds(c * 16, 16)] = zeros16
        return 0

    lax.fori_loop(0, SIDX_BUF // 16, zinit, 0)

    def chunk(c, _):
        tv = rng_v[pl.ds(c * 16, 16)]
        nodes = base + c * 16 + lax.iota(jnp.int32, 16)
        valid = tv < S
        slot = jnp.where(valid, tv, S)
        plsc.store_scatter(sbuf_v, [slot], nodes, mask=valid)
        return 0

    lax.fori_loop(0, NODES_PER // 16, chunk, 0)
    pltpu.sync_copy(sbuf_v, out_hbm.at[wid])


def _make_sidx_kernel():
    return functools.partial(
        pl.kernel,
        mesh=plsc.VectorSubcoreMesh(core_axis_name="c", subcore_axis_name="s"),
        compiler_params=pltpu.CompilerParams(needs_layout_passes=False),
        out_type=jax.ShapeDtypeStruct((NW, SIDX_BUF), jnp.int32),
        scratch_types=[
            pltpu.VMEM((NODES_PER,), jnp.int32),
            pltpu.VMEM((SIDX_BUF,), jnp.int32),
        ],
    )(_sidx_body)


def _edge_body(src_hbm, dst_hbm, tbl_hbm, zt_hbm, dsum_out, csum_out,
               tbl_v, zt_v, s0, d0, s1, d1, out_v, sem0, sem1):
    cid = lax.axis_index("c")
    sid = lax.axis_index("s")
    wid = sid * NC + cid
    pltpu.sync_copy(tbl_hbm, tbl_v)
    pltpu.sync_copy(zt_hbm, zt_v)
    base = wid * E_PER

    zero16 = jnp.zeros((16,), jnp.float32)

    def lookup(sv, dv, off):
        s16 = sv[pl.ds(off, 16)]
        d16 = dv[pl.ds(off, 16)]
        pw = plsc.load_gather(tbl_v, [lax.shift_right_logical(s16, 1)])
        p = lax.shift_right_logical(pw, (s16 & 1) << 4) & 0xFFFF
        qw = plsc.load_gather(tbl_v, [lax.shift_right_logical(d16, 1)])
        q = lax.shift_right_logical(qw, (d16 & 1) << 4) & 0xFFFF
        return p, q, (p < S) & (q < S)

    def pair_dist(p, q, valid):
        pc = jnp.where(valid, p, 0)
        qc = jnp.where(valid, q, 0)
        a2 = zero16
        for d in range(LATENT_DIM):
            zp = plsc.load_gather(zt_v, [pc + d * S])
            zq = plsc.load_gather(zt_v, [qc + d * S])
            df = zp - zq + EPS
            a2 = a2 + df * df
        dist = _nsqrt(a2)
        return (jnp.where(valid, dist, 0.0), jnp.where(valid, 1.0, 0.0))

    def make_chunkw(sv, dv):
        def chunkw(c, carry):
            dacc, cacc = carry
            groups = [lookup(sv, dv, c * 64 + 16 * g) for g in range(4)]
            nv = groups[0][2].astype(jnp.int32)
            for g in range(1, 4):
                nv = nv + groups[g][2].astype(jnp.int32)
            nvs = lax.reduce_max(plsc.all_reduce_population_count(nv > 0),
                                 axes=(0,))

            def heavy(_):
                dd, cc = zero16, zero16
                for p, q, v in groups:
                    dg, cg = pair_dist(p, q, v)
                    dd = dd + dg
                    cc = cc + cg
                return (dd, cc)

            def light(_):
                return (zero16, zero16)

            dd, cc = lax.cond(nvs > 0, heavy, light, 0)
            return (dacc + dd, cacc + cc)
        return chunkw

    def tail16(sv, dv, carry):
        dacc, cacc = carry
        p, q, valid = lookup(sv, dv, E_BLK - 16)
        nv = plsc.all_reduce_population_count(valid)
        nvs = lax.reduce_max(nv, axes=(0,))

        def heavy(_):
            return pair_dist(p, q, valid)

        def light(_):
            return (zero16, zero16)

        dd, cc = lax.cond(nvs > 0, heavy, light, 0)
        return (dacc + dd, cacc + cc)

    bufs = ((s0, d0, sem0), (s1, d1, sem1))
    pltpu.async_copy(src_hbm.at[pl.ds(base, E_BLK)], s0, sem0)
    pltpu.async_copy(dst_hbm.at[pl.ds(base, E_BLK)], d0, sem0)
    carry = (zero16, zero16)
    for b in range(N_BLK):
        sv, dv, sem = bufs[b % 2]
        if b + 1 < N_BLK:
            nsv, ndv, nsem = bufs[(b + 1) % 2]
            noff = base + (b + 1) * E_BLK
            pltpu.async_copy(src_hbm.at[pl.ds(noff, E_BLK)], nsv, nsem)
            pltpu.async_copy(dst_hbm.at[pl.ds(noff, E_BLK)], ndv, nsem)
        off = base + b * E_BLK
        pltpu.make_async_copy(src_hbm.at[pl.ds(off, E_BLK)], sv, sem).wait()
        pltpu.make_async_copy(dst_hbm.at[pl.ds(off, E_BLK)], dv, sem).wait()
        carry = lax.fori_loop(0, (E_BLK - 16) // 64, make_chunkw(sv, dv), carry)
        carry = tail16(sv, dv, carry)
    dacc, cacc = carry
    out_v[...] = dacc
    pltpu.sync_copy(out_v, dsum_out.at[wid])
    out_v[...] = cacc
    pltpu.sync_copy(out_v, csum_out.at[wid])


def _make_edge_kernel():
    return functools.partial(
        pl.kernel,
        mesh=plsc.VectorSubcoreMesh(core_axis_name="c", subcore_axis_name="s"),
        compiler_params=pltpu.CompilerParams(needs_layout_passes=False),
        out_type=[
            jax.ShapeDtypeStruct((NW, 16), jnp.float32),
            jax.ShapeDtypeStruct((NW, 16), jnp.float32),
        ],
        scratch_types=[
            pltpu.VMEM((N_NODES // 2,), jnp.int32),
            pltpu.VMEM((S * LATENT_DIM,), jnp.float32),
            pltpu.VMEM((E_BLK,), jnp.int32),
            pltpu.VMEM((E_BLK,), jnp.int32),
            pltpu.VMEM((E_BLK,), jnp.int32),
            pltpu.VMEM((E_BLK,), jnp.int32),
            pltpu.VMEM((16,), jnp.float32),
            pltpu.SemaphoreType.DMA,
            pltpu.SemaphoreType.DMA,
        ],
    )(_edge_body)


def _pdist_body(zi_ref, za_ref, out_ref):
    zi = zi_ref[...]
    za = za_ref[...]
    acc = jnp.zeros((128, S), jnp.float32)
    for d in range(LATENT_DIM):
        df = zi[:, d][:, None] - za[:, d][None, :] + EPS
        acc = acc + df * df
    mat = jnp.exp(-jnp.sqrt(acc))
    out_ref[...] = jnp.full((1, 1, 128), jnp.sum(mat), jnp.float32)


_pdist_call = pl.pallas_call(
    _pdist_body,
    grid=(8,),
    in_specs=[
        pl.BlockSpec((128, LATENT_DIM), lambda i: (i, 0)),
        pl.BlockSpec((S, LATENT_DIM), lambda i: (0, 0)),
    ],
    out_specs=pl.BlockSpec((1, 1, 128), lambda i: (i, 0, 0)),
    out_shape=jax.ShapeDtypeStruct((8, 1, 128), jnp.float32),
)


@functools.lru_cache(maxsize=1)
def _gumbel_pad():
    skey = jax.random.key(42)
    u = jax.random.uniform(skey, (N_NODES,), minval=1e-9, maxval=1.0)
    gumbel = -jnp.log(-jnp.log(u))
    return jnp.pad(gumbel, (0, NP - N_NODES)).reshape(ROWS, 128)


def kernel(latent_Z, alpha, sampling_weights, edge_index, sample_size):
    w_pad = jnp.pad(sampling_weights, (0, NP - N_NODES),
                    constant_values=1e-30).reshape(ROWS, 128)
    g_pad = _gumbel_pad()

    tbl2d = _thresh_call(w_pad, g_pad)
    tbl = tbl2d.reshape(-1)[:N_NODES]
    tbl_packed = tbl[0::2] | (tbl[1::2] << 16)

    tbl_sc = jnp.pad(tbl, (0, NODES_PAD - N_NODES), constant_values=SENT)
    sidx_parts = _make_sidx_kernel()(tbl_sc)
    sample_idx = jnp.sum(sidx_parts, axis=0)[:S]
    Zs = latent_Z[sample_idx]
    zt_flat = Zs.T.reshape(-1)

    blocksums = _pdist_call(Zs, Zs)
    mat_total = jnp.sum(blocksums[:, 0, 0])
    diag_const = float(np.exp(-np.sqrt(np.sum(
        np.full((LATENT_DIM,), EPS, np.float32) ** 2, dtype=np.float32))))
    offdiag = mat_total - S * diag_const
    e1 = jnp.exp(jnp.float32(1.0))
    z_pdist1 = jnp.exp(alpha[0]) * (0.5 * (e1 * e1 * offdiag))

    dsum, csum = _make_edge_kernel()(
        edge_index[0], edge_index[1], tbl_packed, zt_flat)
    sum_d = jnp.sum(dsum)
    n_valid = jnp.sum(csum)
    z_pdist2 = alpha[0] * n_valid - sum_d

    out = z_pdist2 - z_pdist1
    return jnp.full((1, 1), out, jnp.float32)


# speedup vs baseline: 166.7856x; 1.0387x over previous
import functools

import numpy as np
import jax
import jax.numpy as jnp
from jax import lax
from jax.experimental import pallas as pl
from jax.experimental.pallas import tpu as pltpu
from jax.experimental.pallas import tpu_sc as plsc

N_NODES = 100000
N_EDGES = 1600000
LATENT_DIM = 32
S = 1024
EPS = 1e-6
SENT = 0xFFFF

ROWS = 782
NP = ROWS * 128

try:
    _info = plsc.get_sparse_core_info()
    NC, NS = _info.num_cores, _info.num_subcores
except Exception:
    NC, NS = 2, 16
NW = NC * NS
E_PER = N_EDGES // NW
E_BLK = 10000
N_BLK = E_PER // E_BLK
N_CHUNK = E_BLK // 16

NODES_PER = 3136
NODES_PAD = NODES_PER * NW
SIDX_BUF = 1040


def _nsqrt(x):
    i = lax.bitcast_convert_type(x, jnp.int32)
    y = lax.bitcast_convert_type((i >> 1) + 0x1FBD1DF5, jnp.float32)
    for _ in range(4):
        y = 0.5 * (y + x / y)
    return y


def _cumsum_lanes(x):
    for k in (1, 2, 4, 8, 16, 32, 64):
        x = x + jnp.pad(x, ((0, 0), (k, 0)))[:, :-k]
    return x


def _cumsum_rows(x):
    for k in (1, 2, 4, 8, 16, 32, 64, 128, 256, 512):
        x = x + jnp.pad(x, ((k, 0), (0, 0)))[:-k, :]
    return x


def _thresh_body(w_ref, g_ref, tbl_ref):
    s = jnp.log(w_ref[...]) + g_ref[...]
    i = lax.bitcast_convert_type(s, jnp.int32)
    key = jnp.where(i >= 0, i, i ^ 0x7FFFFFFF)
    cnt_pos = jnp.sum((key >= 0).astype(jnp.int32))
    prefix0 = jnp.where(cnt_pos >= S, 0, jnp.int32(-2147483648))

    def bit_body(t, prefix):
        cand = prefix | lax.shift_left(jnp.int32(1), 30 - t)
        c = jnp.sum((key >= cand).astype(jnp.int32))
        return jnp.where(c >= S, cand, prefix)

    v = lax.fori_loop(0, 31, bit_body, prefix0)
    c_gt = jnp.sum((key > v).astype(jnp.int32))
    need = S - c_gt
    eq32 = (key == v).astype(jnp.int32)
    rowc = _cumsum_lanes(eq32)
    rowtot = rowc[:, 127:128]
    rowoff = _cumsum_rows(rowtot) - rowtot
    excl_eq = rowc - eq32 + rowoff
    mask = (key > v) | ((eq32 > 0) & (excl_eq < need))
    m32 = mask.astype(jnp.int32)
    rc2 = _cumsum_lanes(m32)
    rt2 = rc2[:, 127:128]
    ro2 = _cumsum_rows(rt2) - rt2
    pos = rc2 - m32 + ro2
    tbl_ref[...] = jnp.where(mask, pos, SENT)


_thresh_call = pl.pallas_call(
    _thresh_body,
    in_specs=[
        pl.BlockSpec((ROWS, 128), lambda: (0, 0)),
        pl.BlockSpec((ROWS, 128), lambda: (0, 0)),
    ],
    out_specs=pl.BlockSpec((ROWS, 128), lambda: (0, 0)),
    out_shape=jax.ShapeDtypeStruct((ROWS, 128), jnp.int32),
)


def _sidx_body(tbl_hbm, out_hbm, rng_v, sbuf_v):
    cid = lax.axis_index("c")
    sid = lax.axis_index("s")
    wid = sid * NC + cid
    base = wid * NODES_PER
    pltpu.sync_copy(tbl_hbm.at[pl.ds(base, NODES_PER)], rng_v)
    zeros16 = jnp.zeros((16,), jnp.int32)

    def zinit(c, _):
        sbuf_v[pl.ds(c * 16, 16)] = zeros16
        return 0

    lax.fori_loop(0, SIDX_BUF // 16, zinit, 0)

    def chunk(c, _):
        tv = rng_v[pl.ds(c * 16, 16)]
        nodes = base + c * 16 + lax.iota(jnp.int32, 16)
        valid = tv < S
        slot = jnp.where(valid, tv, S)
        plsc.store_scatter(sbuf_v, [slot], nodes, mask=valid)
        return 0

    lax.fori_loop(0, NODES_PER // 16, chunk, 0)
    pltpu.sync_copy(sbuf_v, out_hbm.at[wid])


def _make_sidx_kernel():
    return functools.partial(
        pl.kernel,
        mesh=plsc.VectorSubcoreMesh(core_axis_name="c", subcore_axis_name="s"),
        compiler_params=pltpu.CompilerParams(needs_layout_passes=False),
        out_type=jax.ShapeDtypeStruct((NW, SIDX_BUF), jnp.int32),
        scratch_types=[
            pltpu.VMEM((NODES_PER,), jnp.int32),
            pltpu.VMEM((SIDX_BUF,), jnp.int32),
        ],
    )(_sidx_body)


def _edge_body(src_hbm, dst_hbm, tbl_hbm, zt_hbm, dsum_out, csum_out,
               tbl_v, zt_v, s0, d0, s1, d1, out_v, sem0, sem1):
    cid = lax.axis_index("c")
    sid = lax.axis_index("s")
    wid = sid * NC + cid
    pltpu.sync_copy(tbl_hbm, tbl_v)
    pltpu.sync_copy(zt_hbm, zt_v)
    base = wid * E_PER

    zero16 = jnp.zeros((16,), jnp.float32)

    def lookup(sv, dv, off):
        s16 = sv[pl.ds(off, 16)]
        d16 = dv[pl.ds(off, 16)]
        pw = plsc.load_gather(tbl_v, [lax.shift_right_logical(s16, 1)])
        p = lax.shift_right_logical(pw, (s16 & 1) << 4) & 0xFFFF
        qw = plsc.load_gather(tbl_v, [lax.shift_right_logical(d16, 1)])
        q = lax.shift_right_logical(qw, (d16 & 1) << 4) & 0xFFFF
        return p, q, (p < S) & (q < S)

    def pair_dist(p, q, valid):
        pc = jnp.where(valid, p, 0)
        qc = jnp.where(valid, q, 0)
        a2 = zero16
        for d in range(LATENT_DIM):
            zp = plsc.load_gather(zt_v, [pc + d * S])
            zq = plsc.load_gather(zt_v, [qc + d * S])
            df = zp - zq + EPS
            a2 = a2 + df * df
        dist = _nsqrt(a2)
        return (jnp.where(valid, dist, 0.0), jnp.where(valid, 1.0, 0.0))

    def make_chunkw(sv, dv):
        def chunkw(c, carry):
            dacc, cacc = carry
            groups = [lookup(sv, dv, c * 128 + 16 * g) for g in range(8)]
            nv = groups[0][2].astype(jnp.int32)
            for g in range(1, 8):
                nv = nv + groups[g][2].astype(jnp.int32)
            nvs = lax.reduce_max(plsc.all_reduce_population_count(nv > 0),
                                 axes=(0,))

            def heavy(_):
                dd, cc = zero16, zero16
                for p, q, v in groups:
                    dg, cg = pair_dist(p, q, v)
                    dd = dd + dg
                    cc = cc + cg
                return (dd, cc)

            def light(_):
                return (zero16, zero16)

            dd, cc = lax.cond(nvs > 0, heavy, light, 0)
            return (dacc + dd, cacc + cc)
        return chunkw

    def tail16(sv, dv, carry):
        dacc, cacc = carry
        p, q, valid = lookup(sv, dv, E_BLK - 16)
        nv = plsc.all_reduce_population_count(valid)
        nvs = lax.reduce_max(nv, axes=(0,))

        def heavy(_):
            return pair_dist(p, q, valid)

        def light(_):
            return (zero16, zero16)

        dd, cc = lax.cond(nvs > 0, heavy, light, 0)
        return (dacc + dd, cacc + cc)

    bufs = ((s0, d0, sem0), (s1, d1, sem1))
    pltpu.async_copy(src_hbm.at[pl.ds(base, E_BLK)], s0, sem0)
    pltpu.async_copy(dst_hbm.at[pl.ds(base, E_BLK)], d0, sem0)
    carry = (zero16, zero16)
    for b in range(N_BLK):
        sv, dv, sem = bufs[b % 2]
        if b + 1 < N_BLK:
            nsv, ndv, nsem = bufs[(b + 1) % 2]
            noff = base + (b + 1) * E_BLK
            pltpu.async_copy(src_hbm.at[pl.ds(noff, E_BLK)], nsv, nsem)
            pltpu.async_copy(dst_hbm.at[pl.ds(noff, E_BLK)], ndv, nsem)
        off = base + b * E_BLK
        pltpu.make_async_copy(src_hbm.at[pl.ds(off, E_BLK)], sv, sem).wait()
        pltpu.make_async_copy(dst_hbm.at[pl.ds(off, E_BLK)], dv, sem).wait()
        carry = lax.fori_loop(0, (E_BLK - 16) // 128, make_chunkw(sv, dv), carry)
        carry = tail16(sv, dv, carry)
    dacc, cacc = carry
    out_v[...] = dacc
    pltpu.sync_copy(out_v, dsum_out.at[wid])
    out_v[...] = cacc
    pltpu.sync_copy(out_v, csum_out.at[wid])


def _make_edge_kernel():
    return functools.partial(
        pl.kernel,
        mesh=plsc.VectorSubcoreMesh(core_axis_name="c", subcore_axis_name="s"),
        compiler_params=pltpu.CompilerParams(needs_layout_passes=False),
        out_type=[
            jax.ShapeDtypeStruct((NW, 16), jnp.float32),
            jax.ShapeDtypeStruct((NW, 16), jnp.float32),
        ],
        scratch_types=[
            pltpu.VMEM((N_NODES // 2,), jnp.int32),
            pltpu.VMEM((S * LATENT_DIM,), jnp.float32),
            pltpu.VMEM((E_BLK,), jnp.int32),
            pltpu.VMEM((E_BLK,), jnp.int32),
            pltpu.VMEM((E_BLK,), jnp.int32),
            pltpu.VMEM((E_BLK,), jnp.int32),
            pltpu.VMEM((16,), jnp.float32),
            pltpu.SemaphoreType.DMA,
            pltpu.SemaphoreType.DMA,
        ],
    )(_edge_body)


def _pdist_body(zi_ref, za_ref, out_ref):
    zi = zi_ref[...]
    za = za_ref[...]
    acc = jnp.zeros((128, S), jnp.float32)
    for d in range(LATENT_DIM):
        df = zi[:, d][:, None] - za[:, d][None, :] + EPS
        acc = acc + df * df
    mat = jnp.exp(-jnp.sqrt(acc))
    out_ref[...] = jnp.full((1, 1, 128), jnp.sum(mat), jnp.float32)


_pdist_call = pl.pallas_call(
    _pdist_body,
    grid=(8,),
    in_specs=[
        pl.BlockSpec((128, LATENT_DIM), lambda i: (i, 0)),
        pl.BlockSpec((S, LATENT_DIM), lambda i: (0, 0)),
    ],
    out_specs=pl.BlockSpec((1, 1, 128), lambda i: (i, 0, 0)),
    out_shape=jax.ShapeDtypeStruct((8, 1, 128), jnp.float32),
)


@functools.lru_cache(maxsize=1)
def _gumbel_pad():
    skey = jax.random.key(42)
    u = jax.random.uniform(skey, (N_NODES,), minval=1e-9, maxval=1.0)
    gumbel = -jnp.log(-jnp.log(u))
    return jnp.pad(gumbel, (0, NP - N_NODES)).reshape(ROWS, 128)


def kernel(latent_Z, alpha, sampling_weights, edge_index, sample_size):
    w_pad = jnp.pad(sampling_weights, (0, NP - N_NODES),
                    constant_values=1e-30).reshape(ROWS, 128)
    g_pad = _gumbel_pad()

    tbl2d = _thresh_call(w_pad, g_pad)
    tbl = tbl2d.reshape(-1)[:N_NODES]
    tbl_packed = tbl[0::2] | (tbl[1::2] << 16)

    tbl_sc = jnp.pad(tbl, (0, NODES_PAD - N_NODES), constant_values=SENT)
    sidx_parts = _make_sidx_kernel()(tbl_sc)
    sample_idx = jnp.sum(sidx_parts, axis=0)[:S]
    Zs = latent_Z[sample_idx]
    zt_flat = Zs.T.reshape(-1)

    blocksums = _pdist_call(Zs, Zs)
    mat_total = jnp.sum(blocksums[:, 0, 0])
    diag_const = float(np.exp(-np.sqrt(np.sum(
        np.full((LATENT_DIM,), EPS, np.float32) ** 2, dtype=np.float32))))
    offdiag = mat_total - S * diag_const
    e1 = jnp.exp(jnp.float32(1.0))
    z_pdist1 = jnp.exp(alpha[0]) * (0.5 * (e1 * e1 * offdiag))

    dsum, csum = _make_edge_kernel()(
        edge_index[0], edge_index[1], tbl_packed, zt_flat)
    sum_d = jnp.sum(dsum)
    n_valid = jnp.sum(csum)
    z_pdist2 = alpha[0] * n_valid - sum_d

    out = z_pdist2 - z_pdist1
    return jnp.full((1, 1), out, jnp.float32)
